# Initial kernel scaffold; baseline (speedup 1.0000x reference)
import jax
import jax.numpy as jnp
from jax.experimental import pallas as pl


def kernel(inputs, edge_index, edge_weight, weight, bias):
    raise NotImplementedError("write your pallas kernel here")


# speedup vs baseline: 2.9967x; 2.9967x over previous
import functools

import jax
import jax.numpy as jnp
from jax import lax
from jax.experimental import pallas as pl
from jax.experimental.pallas import tpu as pltpu
from jax.experimental.pallas import tpu_sc as plsc


def _mm_body(x_ref, w_ref, o_ref):
    res = jnp.dot(x_ref[...], w_ref[...], preferred_element_type=jnp.float32)
    h = res.shape[1] // 2
    o_ref[0] = res[:, :h]
    o_ref[1] = res[:, h:]


def _matmul_split(x, w):
    n, k = x.shape
    _, m = w.shape
    br = 400
    assert n % br == 0 and m % 2 == 0
    return pl.pallas_call(
        _mm_body,
        grid=(n // br,),
        in_specs=[
            pl.BlockSpec((br, k), lambda i: (i, 0)),
            pl.BlockSpec((k, m), lambda i: (0, 0)),
        ],
        out_specs=pl.BlockSpec((2, br, m // 2), lambda i: (0, i, 0)),
        out_shape=jax.ShapeDtypeStruct((2, n, m // 2), jnp.float32),
    )(x, w)


def _comb_body(p_ref, b_ref, o_ref):
    h = p_ref.shape[2]
    o_ref[:, :h] = p_ref[0] + b_ref[:, :h]
    o_ref[:, h:] = p_ref[1] + b_ref[:, h:]


def _combine(partials, bias2d, n):
    _, npad, h = partials.shape
    br = 400
    assert n % br == 0 and npad >= n
    return pl.pallas_call(
        _comb_body,
        grid=(n // br,),
        in_specs=[
            pl.BlockSpec((2, br, h), lambda i: (0, i, 0)),
            pl.BlockSpec((1, 2 * h), lambda i: (0, 0)),
        ],
        out_specs=pl.BlockSpec((br, 2 * h), lambda i: (i, 0)),
        out_shape=jax.ShapeDtypeStruct((n, 2 * h), jnp.float32),
    )(partials, bias2d)


def _sc_aggregate(sup2, src2, dst2, w2, zeros):
    _, n, h = sup2.shape
    npad = zeros.shape[0]
    ns_, nk, ch = src2.shape
    info = plsc.get_sparse_core_info()
    nc, ns = info.num_cores, info.num_subcores
    assert ns_ == ns and h % 16 == 0 and ch % 8 == 0
    assert npad % (8 * ns) == 0 and npad >= n
    rpt = npad // ns

    mesh = plsc.VectorSubcoreMesh(core_axis_name="c", subcore_axis_name="s")

    @functools.partial(
        pl.kernel,
        mesh=mesh,
        compiler_params=pltpu.CompilerParams(use_tc_tiling_on_sc=False),
        out_type=jax.ShapeDtypeStruct((nc, npad, h), jnp.float32),
        scratch_types=[
            pltpu.VMEM((nk, ch), jnp.int32),
            pltpu.VMEM((nk, ch), jnp.int32),
            pltpu.VMEM((nk, ch), jnp.float32),
            pltpu.VMEM((ch, h), jnp.float32),
            pltpu.VMEM_SHARED((npad, h), jnp.float32),
            pltpu.SemaphoreType.DMA,
        ],
    )
    def agg(sup_hbm, src_hbm, dst_hbm, w_hbm, zeros_hbm, out_hbm,
            sidx, didx, wv, rows, acc, sem):
        c = lax.axis_index("c")
        s = lax.axis_index("s")
        r0 = s * rpt

        pltpu.sync_copy(zeros_hbm.at[pl.ds(r0, rpt)],
                        acc.at[pl.ds(r0, rpt)])
        pltpu.sync_copy(src_hbm.at[s], sidx)
        pltpu.sync_copy(dst_hbm.at[s], didx)
        pltpu.sync_copy(w_hbm.at[s], wv)
        plsc.subcore_barrier()

        def chunk_body(k, carry):
            pltpu.async_copy(sup_hbm.at[c].at[sidx.at[k]], rows, sem).wait()

            def scale_body(g, inner):
                wg = wv[k, pl.ds(g * 16, 16)]
                for l in range(16):
                    w = wg[l]
                    e = g * 16 + l
                    for j in range(h // 16):
                        sl = pl.ds(j * 16, 16)
                        rows[e, sl] = rows[e, sl] * w
                return inner

            lax.fori_loop(0, ch // 16, scale_body, 0)
            pltpu.sync_copy(rows, acc.at[didx.at[k]], add=True)
            return carry

        lax.fori_loop(0, nk, chunk_body, 0)
        plsc.subcore_barrier()
        pltpu.sync_copy(acc.at[pl.ds(r0, rpt)],
                        out_hbm.at[c, pl.ds(r0, rpt)])

    return agg(sup2, src2, dst2, w2, zeros)


def kernel(inputs, edge_index, edge_weight, weight, bias):
    n, d_in = inputs.shape
    e = edge_index.shape[1]
    d_out = weight.shape[1]

    ns = 16
    ch = 80
    assert e % (ns * ch) == 0
    nk = e // (ns * ch)

    sup2 = _matmul_split(inputs, weight)

    src2 = edge_index[0].reshape(ns, nk, ch)
    dst2 = edge_index[1].reshape(ns, nk, ch)
    w2 = edge_weight.reshape(ns, nk, ch)

    npad = ((n + ns * 8 - 1) // (ns * 8)) * (ns * 8)
    zeros = jnp.zeros((npad, d_out // 2), jnp.float32)

    partials = _sc_aggregate(sup2, src2, dst2, w2, zeros)

    return _combine(partials, bias.reshape(1, d_out), n)


# speedup vs baseline: 7.8940x; 2.6343x over previous
import functools

import jax
import jax.numpy as jnp
from jax import lax
from jax.experimental import pallas as pl
from jax.experimental.pallas import tpu as pltpu
from jax.experimental.pallas import tpu_sc as plsc


def _mm_body(x_ref, w_ref, o_ref):
    res = jnp.dot(x_ref[...], w_ref[...], preferred_element_type=jnp.float32)
    h = res.shape[1] // 2
    o_ref[0] = res[:, :h]
    o_ref[1] = res[:, h:]


def _matmul_split(x, w):
    n, k = x.shape
    _, m = w.shape
    br = 400
    assert n % br == 0 and m % 2 == 0
    return pl.pallas_call(
        _mm_body,
        grid=(n // br,),
        in_specs=[
            pl.BlockSpec((br, k), lambda i: (i, 0)),
            pl.BlockSpec((k, m), lambda i: (0, 0)),
        ],
        out_specs=pl.BlockSpec((2, br, m // 2), lambda i: (0, i, 0)),
        out_shape=jax.ShapeDtypeStruct((2, n, m // 2), jnp.float32),
    )(x, w)


def _comb_body(p_ref, b_ref, o_ref):
    h = p_ref.shape[2]
    o_ref[:, :h] = p_ref[0] + b_ref[:, :h]
    o_ref[:, h:] = p_ref[1] + b_ref[:, h:]


def _combine(partials, bias2d, n):
    _, npad, h = partials.shape
    br = 400
    assert n % br == 0 and npad >= n
    return pl.pallas_call(
        _comb_body,
        grid=(n // br,),
        in_specs=[
            pl.BlockSpec((2, br, h), lambda i: (0, i, 0)),
            pl.BlockSpec((1, 2 * h), lambda i: (0, 0)),
        ],
        out_specs=pl.BlockSpec((br, 2 * h), lambda i: (i, 0)),
        out_shape=jax.ShapeDtypeStruct((n, 2 * h), jnp.float32),
    )(partials, bias2d)


def _sc_aggregate(sup2, src2, dst2, w2, zeros):
    _, n, h = sup2.shape
    npad = zeros.shape[0]
    ns_, nk, ch = src2.shape
    info = plsc.get_sparse_core_info()
    nc, ns = info.num_cores, info.num_subcores
    assert ns_ == ns and h % 16 == 0 and ch % 8 == 0
    assert npad % (8 * ns) == 0 and npad >= n
    rpt = npad // ns

    mesh = plsc.VectorSubcoreMesh(core_axis_name="c", subcore_axis_name="s")

    @functools.partial(
        pl.kernel,
        mesh=mesh,
        compiler_params=pltpu.CompilerParams(use_tc_tiling_on_sc=False),
        out_type=jax.ShapeDtypeStruct((nc, npad, h), jnp.float32),
        scratch_types=[
            pltpu.VMEM((nk, ch), jnp.int32),
            pltpu.VMEM((nk, ch), jnp.int32),
            pltpu.VMEM((nk, ch), jnp.float32),
            pltpu.VMEM((ch, h), jnp.float32),
            pltpu.VMEM((ch, h), jnp.float32),
            pltpu.VMEM_SHARED((npad, h), jnp.float32),
            pltpu.SemaphoreType.DMA,
            pltpu.SemaphoreType.DMA,
        ],
    )
    def agg(sup_hbm, src_hbm, dst_hbm, w_hbm, zeros_hbm, out_hbm,
            sidx, didx, wv, rows0, rows1, acc, sem0, sem1):
        c = lax.axis_index("c")
        s = lax.axis_index("s")
        r0 = s * rpt

        pltpu.sync_copy(zeros_hbm.at[pl.ds(r0, rpt)],
                        acc.at[pl.ds(r0, rpt)])
        pltpu.sync_copy(src_hbm.at[s], sidx)
        pltpu.sync_copy(dst_hbm.at[s], didx)
        pltpu.sync_copy(w_hbm.at[s], wv)
        plsc.subcore_barrier()

        def start_gather(k, rows, sem):
            pltpu.async_copy(sup_hbm.at[c].at[sidx.at[k]], rows, sem)

        def process(k, rows, sem):
            pltpu.make_async_copy(sup_hbm.at[c].at[sidx.at[k]], rows,
                                  sem).wait()

            @plsc.parallel_loop(0, ch // 16, unroll=2)
            def scale_body(g):
                wg = wv[k, pl.ds(g * 16, 16)]
                for l in range(16):
                    w = wg[l]
                    e = g * 16 + l
                    for j in range(h // 16):
                        sl = pl.ds(j * 16, 16)
                        rows[e, sl] = rows[e, sl] * w

            pltpu.sync_copy(rows, acc.at[didx.at[k]], add=True)

        start_gather(0, rows0, sem0)

        def pair_body(k2, carry):
            ka = 2 * k2
            start_gather(ka + 1, rows1, sem1)
            process(ka, rows0, sem0)

            @pl.when(ka + 2 < nk)
            def _():
                start_gather(ka + 2, rows0, sem0)

            process(ka + 1, rows1, sem1)
            return carry

        assert nk % 2 == 0
        lax.fori_loop(0, nk // 2, pair_body, 0)
        plsc.subcore_barrier()
        pltpu.sync_copy(acc.at[pl.ds(r0, rpt)],
                        out_hbm.at[c, pl.ds(r0, rpt)])

    return agg(sup2, src2, dst2, w2, zeros)


def kernel(inputs, edge_index, edge_weight, weight, bias):
    n, d_in = inputs.shape
    e = edge_index.shape[1]
    d_out = weight.shape[1]

    ns = 16
    ch = 80
    assert e % (ns * ch) == 0
    nk = e // (ns * ch)

    sup2 = _matmul_split(inputs, weight)

    src2 = edge_index[0].reshape(ns, nk, ch)
    dst2 = edge_index[1].reshape(ns, nk, ch)
    w2 = edge_weight.reshape(ns, nk, ch)

    npad = ((n + ns * 8 - 1) // (ns * 8)) * (ns * 8)
    zeros = jnp.zeros((npad, d_out // 2), jnp.float32)

    partials = _sc_aggregate(sup2, src2, dst2, w2, zeros)

    return _combine(partials, bias.reshape(1, d_out), n)


# speedup vs baseline: 10.2131x; 1.2938x over previous
import functools

import jax
import jax.numpy as jnp
from jax import lax
from jax.experimental import pallas as pl
from jax.experimental.pallas import tpu as pltpu
from jax.experimental.pallas import tpu_sc as plsc


def _mm_body(p_ref, w_ref, b_ref, o_ref):
    agg = p_ref[0].astype(jnp.float32) + p_ref[1].astype(jnp.float32)
    o_ref[...] = (jnp.dot(agg, w_ref[...], preferred_element_type=jnp.float32)
                  + b_ref[...])


def _matmul_bias(partials, w, bias2d, n):
    _, np_, d = partials.shape
    _, m = w.shape
    br = 400
    assert n % br == 0 and np_ >= n
    return pl.pallas_call(
        _mm_body,
        grid=(n // br,),
        in_specs=[
            pl.BlockSpec((2, br, d), lambda i: (0, i, 0)),
            pl.BlockSpec((d, m), lambda i: (0, 0)),
            pl.BlockSpec((1, m), lambda i: (0, 0)),
        ],
        out_specs=pl.BlockSpec((br, m), lambda i: (i, 0)),
        out_shape=jax.ShapeDtypeStruct((n, m), jnp.float32),
    )(partials, w, bias2d)


def _sc_aggregate(sup2, epk, zeros):
    _, n, h = sup2.shape
    nw, _, nk, ch = epk.shape
    info = plsc.get_sparse_core_info()
    nc, ns = info.num_cores, info.num_subcores
    assert nw == nc * ns and h % 32 == 0 and ch % 8 == 0
    assert zeros.shape == (n, h)
    assert 14 * 624 + 2 * 632 == n

    mesh = plsc.VectorSubcoreMesh(core_axis_name="c", subcore_axis_name="s")

    @functools.partial(
        pl.kernel,
        mesh=mesh,
        compiler_params=pltpu.CompilerParams(use_tc_tiling_on_sc=False,
                                             needs_layout_passes=False),
        out_type=jax.ShapeDtypeStruct((nc, n, h), jnp.bfloat16),
        scratch_types=[
            pltpu.VMEM((nk, ch), jnp.int32),
            pltpu.VMEM((nk, ch), jnp.int32),
            pltpu.VMEM((nk, ch), jnp.int32),
            [pltpu.VMEM((ch, h), jnp.bfloat16)] * 4,
            pltpu.VMEM_SHARED((n, h), jnp.bfloat16),
            [pltpu.SemaphoreType.DMA] * 4,
            [pltpu.SemaphoreType.DMA] * 4,
        ],
    )
    def agg(sup_hbm, epk_hbm, zeros_hbm, out_hbm,
            sidx, didx, wv, rows, acc, gsem, ssem):
        c = lax.axis_index("c")
        s = lax.axis_index("s")
        tid = c * ns + s

        @pl.when(s < 14)
        def _():
            r0 = s * 624
            pltpu.sync_copy(zeros_hbm.at[pl.ds(r0, 624)],
                            acc.at[pl.ds(r0, 624)])

        @pl.when(s >= 14)
        def _():
            r0 = 14 * 624 + (s - 14) * 632
            pltpu.sync_copy(zeros_hbm.at[pl.ds(r0, 632)],
                            acc.at[pl.ds(r0, 632)])

        pltpu.sync_copy(epk_hbm.at[tid, 0], sidx)
        pltpu.sync_copy(epk_hbm.at[tid, 1], didx)
        pltpu.sync_copy(epk_hbm.at[tid, 2], wv)
        plsc.subcore_barrier()

        nbuf = len(rows)

        def start_gather(k, b):
            pltpu.async_copy(sup_hbm.at[0].at[sidx.at[k]], rows[b], gsem[b])

        def wait_gather(k, b):
            pltpu.make_async_copy(sup_hbm.at[0].at[sidx.at[k]], rows[b],
                                  gsem[b]).wait()

        def scale(k, b):
            @plsc.parallel_loop(0, ch // 16, unroll=2)
            def scale_body(g):
                wg = wv[k, pl.ds(g * 16, 16)]
                for l in range(16):
                    wsplat = plsc.bitcast(jnp.broadcast_to(wg[l], (16,)),
                                          jnp.bfloat16)
                    e = g * 16 + l
                    for j in range(h // 32):
                        sl = pl.ds(j * 32, 32)
                        rows[b][e, sl] = rows[b][e, sl] * wsplat

        def start_scatter(k, b):
            pltpu.async_copy(rows[b], acc.at[didx.at[k]], ssem[b], add=True)

        def wait_scatter(k, b):
            pltpu.make_async_copy(rows[b], acc.at[didx.at[k]],
                                  ssem[b]).wait()

        for b in range(nbuf):
            start_gather(b, b)

        nq, rem = divmod(nk, nbuf)

        def ring_body(q, carry):
            kx = q * nbuf
            for b in range(nbuf):
                wait_gather(kx + b, b)
                scale(kx + b, b)
                start_scatter(kx + b, b)
            for b in range(nbuf):
                wait_scatter(kx + b, b)

                @pl.when(kx + b + nbuf < nk)
                def _(b=b):
                    start_gather(kx + b + nbuf, b)

            return carry

        lax.fori_loop(0, nq, ring_body, 0)
        for b in range(rem):
            kx = nq * nbuf + b
            wait_gather(kx, b)
            scale(kx, b)
            start_scatter(kx, b)
        for b in range(rem):
            wait_scatter(nq * nbuf + b, b)
        plsc.subcore_barrier()

        @pl.when(s < 14)
        def _():
            r0 = s * 624
            pltpu.sync_copy(acc.at[pl.ds(r0, 624)],
                            out_hbm.at[c, pl.ds(r0, 624)])

        @pl.when(s >= 14)
        def _():
            r0 = 14 * 624 + (s - 14) * 632
            pltpu.sync_copy(acc.at[pl.ds(r0, 632)],
                            out_hbm.at[c, pl.ds(r0, 632)])

    return agg(sup2, epk, zeros)


def kernel(inputs, edge_index, edge_weight, weight, bias):
    n, d_in = inputs.shape
    e = edge_index.shape[1]
    d_out = weight.shape[1]

    nw = 32
    ch = 80
    assert e % (nw * ch) == 0
    nk = e // (nw * ch)

    xbf = inputs.astype(jnp.bfloat16)
    sup2 = jnp.stack([xbf, jnp.zeros((n, d_in), jnp.bfloat16)])

    src2 = edge_index[0].reshape(nw, 1, nk, ch)
    dst2 = edge_index[1].reshape(nw, 1, nk, ch)
    wb = lax.bitcast_convert_type(edge_weight.astype(jnp.bfloat16),
                                  jnp.uint16).astype(jnp.int32)
    wbits = (wb | (wb << 16)).reshape(nw, 1, nk, ch)
    pad2 = jnp.zeros((nw, 1, nk, ch), jnp.int32)
    epk = jnp.concatenate([src2, dst2, wbits, pad2], axis=1)

    zeros = jnp.zeros((n, d_in), jnp.bfloat16)

    partials = _sc_aggregate(sup2, epk, zeros)

    return _matmul_bias(partials, weight, bias.reshape(1, d_out), n)


# speedup vs baseline: 11.9484x; 1.1699x over previous
import functools

import jax
import jax.numpy as jnp
from jax import lax
from jax.experimental import pallas as pl
from jax.experimental.pallas import tpu as pltpu
from jax.experimental.pallas import tpu_sc as plsc


def _mm_body(p_ref, w_ref, b_ref, o_ref):
    agg = (p_ref[0].astype(jnp.float32)
           + p_ref[1].astype(jnp.float32)).astype(jnp.bfloat16)
    o_ref[...] = (jnp.dot(agg, w_ref[...].astype(jnp.bfloat16),
                          preferred_element_type=jnp.float32)
                  + b_ref[...])


def _matmul_bias(partials, w, bias2d, n):
    _, np_, d = partials.shape
    _, m = w.shape
    br = 400
    assert n % br == 0 and np_ >= n
    return pl.pallas_call(
        _mm_body,
        grid=(n // br,),
        in_specs=[
            pl.BlockSpec((2, br, d), lambda i: (0, i, 0)),
            pl.BlockSpec((d, m), lambda i: (0, 0)),
            pl.BlockSpec((1, m), lambda i: (0, 0)),
        ],
        out_specs=pl.BlockSpec((br, m), lambda i: (i, 0)),
        out_shape=jax.ShapeDtypeStruct((n, m), jnp.float32),
    )(partials, w, bias2d)


def _sc_aggregate(sup, epk):
    n, h = sup.shape
    nw, _, nk, ch = epk.shape
    info = plsc.get_sparse_core_info()
    nc, ns = info.num_cores, info.num_subcores
    assert nw == nc * ns and h % 32 == 0 and ch % 8 == 0 and n < 2 ** 16
    assert 14 * 624 + 2 * 632 == n

    mesh = plsc.VectorSubcoreMesh(core_axis_name="c", subcore_axis_name="s")

    @functools.partial(
        pl.kernel,
        mesh=mesh,
        compiler_params=pltpu.CompilerParams(use_tc_tiling_on_sc=False,
                                             needs_layout_passes=False),
        out_type=jax.ShapeDtypeStruct((nc, n, h), jnp.bfloat16),
        scratch_types=[
            pltpu.VMEM((nk, ch), jnp.int32),
            pltpu.VMEM((nk, ch), jnp.int32),
            pltpu.VMEM((nk, ch), jnp.int32),
            pltpu.VMEM((nk, ch), jnp.int32),
            [pltpu.VMEM((ch, h), jnp.bfloat16)] * 4,
            pltpu.VMEM_SHARED((n, h), jnp.bfloat16),
            [pltpu.SemaphoreType.DMA] * 4,
            [pltpu.SemaphoreType.DMA] * 4,
        ],
    )
    def agg(sup_hbm, epk_hbm, out_hbm,
            sd, sidx, didx, wv, rows, acc, gsem, ssem):
        c = lax.axis_index("c")
        s = lax.axis_index("s")
        tid = c * ns + s

        pltpu.sync_copy(epk_hbm.at[tid, 0], sd)
        pltpu.sync_copy(epk_hbm.at[tid, 1], wv)

        zb = jnp.zeros((32,), jnp.bfloat16)

        @plsc.parallel_loop(0, ch)
        def zero_rows(r):
            for j in range(h // 32):
                rows[0][r, pl.ds(j * 32, 32)] = zb

        def zero_acc(r0, rpt):
            for i in range(rpt // ch):
                pltpu.sync_copy(rows[0], acc.at[pl.ds(r0 + i * ch, ch)])
            t = rpt % ch
            if t:
                pltpu.sync_copy(rows[0].at[pl.ds(0, t)],
                                acc.at[pl.ds(r0 + (rpt // ch) * ch, t)])

        @pl.when(s < 14)
        def _():
            zero_acc(s * 624, 624)

        @pl.when(s >= 14)
        def _():
            zero_acc(14 * 624 + (s - 14) * 632, 632)

        @plsc.parallel_loop(0, nk)
        def unpack_idx(k):
            for g in range(ch // 16):
                sl = pl.ds(g * 16, 16)
                v = sd[k, sl]
                sidx[k, sl] = v & 0xFFFF
                didx[k, sl] = lax.shift_right_logical(v, 16)

        plsc.subcore_barrier()

        nbuf = len(rows)

        def start_gather(k, b):
            pltpu.async_copy(sup_hbm.at[sidx.at[k]], rows[b], gsem[b])

        def wait_gather(k, b):
            pltpu.make_async_copy(sup_hbm.at[sidx.at[k]], rows[b],
                                  gsem[b]).wait()

        def scale(k, b):
            @plsc.parallel_loop(0, ch // 16, unroll=2)
            def scale_body(g):
                wg = wv[k, pl.ds(g * 16, 16)]
                for l in range(16):
                    wsplat = plsc.bitcast(jnp.broadcast_to(wg[l], (16,)),
                                          jnp.bfloat16)
                    e = g * 16 + l
                    for j in range(h // 32):
                        sl = pl.ds(j * 32, 32)
                        rows[b][e, sl] = rows[b][e, sl] * wsplat

        def start_scatter(k, b):
            pltpu.async_copy(rows[b], acc.at[didx.at[k]], ssem[b], add=True)

        def wait_scatter(k, b):
            pltpu.make_async_copy(rows[b], acc.at[didx.at[k]],
                                  ssem[b]).wait()

        for b in range(nbuf):
            start_gather(b, b)

        nq, rem = divmod(nk, nbuf)

        def ring_body(q, carry):
            kx = q * nbuf
            for b in range(nbuf):
                wait_gather(kx + b, b)
                scale(kx + b, b)
                start_scatter(kx + b, b)
            for b in range(nbuf):
                wait_scatter(kx + b, b)

                @pl.when(kx + b + nbuf < nk)
                def _(b=b):
                    start_gather(kx + b + nbuf, b)

            return carry

        lax.fori_loop(0, nq, ring_body, 0)
        for b in range(rem):
            kx = nq * nbuf + b
            wait_gather(kx, b)
            scale(kx, b)
            start_scatter(kx, b)
        for b in range(rem):
            wait_scatter(nq * nbuf + b, b)
        plsc.subcore_barrier()

        @pl.when(s < 14)
        def _():
            r0 = s * 624
            pltpu.sync_copy(acc.at[pl.ds(r0, 624)],
                            out_hbm.at[c, pl.ds(r0, 624)])

        @pl.when(s >= 14)
        def _():
            r0 = 14 * 624 + (s - 14) * 632
            pltpu.sync_copy(acc.at[pl.ds(r0, 632)],
                            out_hbm.at[c, pl.ds(r0, 632)])

    return agg(sup, epk)


def kernel(inputs, edge_index, edge_weight, weight, bias):
    n, d_in = inputs.shape
    e = edge_index.shape[1]
    d_out = weight.shape[1]

    nw = 32
    ch = 80
    assert e % (nw * ch) == 0
    nk = e // (nw * ch)

    xbf = inputs.astype(jnp.bfloat16)

    sd = (edge_index[0] | (edge_index[1] << 16)).reshape(nw, 1, nk, ch)
    wb = lax.bitcast_convert_type(edge_weight.astype(jnp.bfloat16),
                                  jnp.uint16).astype(jnp.int32)
    wbits = (wb | (wb << 16)).reshape(nw, 1, nk, ch)
    epk = jnp.concatenate([sd, wbits], axis=1)

    partials = _sc_aggregate(xbf, epk)

    return _matmul_bias(partials, weight, bias.reshape(1, d_out), n)


# speedup vs baseline: 12.1154x; 1.0140x over previous
import functools

import jax
import jax.numpy as jnp
from jax import lax
from jax.experimental import pallas as pl
from jax.experimental.pallas import tpu as pltpu
from jax.experimental.pallas import tpu_sc as plsc


def _mm_body(p_ref, w_ref, b_ref, o_ref):
    agg = (p_ref[0].astype(jnp.float32)
           + p_ref[1].astype(jnp.float32)).astype(jnp.bfloat16)
    o_ref[...] = (jnp.dot(agg, w_ref[...].astype(jnp.bfloat16),
                          preferred_element_type=jnp.float32)
                  + b_ref[...])


def _matmul_bias(partials, w, bias2d, n):
    _, np_, d = partials.shape
    _, m = w.shape
    br = 400
    assert n % br == 0 and np_ >= n
    return pl.pallas_call(
        _mm_body,
        grid=(n // br,),
        in_specs=[
            pl.BlockSpec((2, br, d), lambda i: (0, i, 0)),
            pl.BlockSpec((d, m), lambda i: (0, 0)),
            pl.BlockSpec((1, m), lambda i: (0, 0)),
        ],
        out_specs=pl.BlockSpec((br, m), lambda i: (i, 0)),
        out_shape=jax.ShapeDtypeStruct((n, m), jnp.float32),
    )(partials, w, bias2d)


def _sc_aggregate(sup, sd_hbm_arr, w_hbm_arr, nk, ch):
    n, h = sup.shape
    nw, ec = sd_hbm_arr.shape
    info = plsc.get_sparse_core_info()
    nc, ns = info.num_cores, info.num_subcores
    assert nw == nc * ns and h % 32 == 0 and ch % 8 == 0 and n < 2 ** 16
    assert ec == nk * ch and w_hbm_arr.shape == (nw, ec)
    assert 14 * 624 + 2 * 632 == n

    mesh = plsc.VectorSubcoreMesh(core_axis_name="c", subcore_axis_name="s")

    @functools.partial(
        pl.kernel,
        mesh=mesh,
        compiler_params=pltpu.CompilerParams(use_tc_tiling_on_sc=False,
                                             needs_layout_passes=False),
        out_type=jax.ShapeDtypeStruct((nc, n, h), jnp.bfloat16),
        scratch_types=[
            pltpu.VMEM((nk * ch,), jnp.int32),
            pltpu.VMEM((nk, ch), jnp.int32),
            pltpu.VMEM((nk, ch), jnp.int32),
            pltpu.VMEM((nk * ch,), jnp.int32),
            [pltpu.VMEM((ch, h), jnp.bfloat16)] * 8,
            pltpu.VMEM_SHARED((n, h), jnp.bfloat16),
            [pltpu.SemaphoreType.DMA] * 8,
            [pltpu.SemaphoreType.DMA] * 8,
        ],
    )
    def agg(sup_hbm, sd_hbm, w_hbm, out_hbm,
            sd, sidx, didx, wv, rows, acc, gsem, ssem):
        c = lax.axis_index("c")
        s = lax.axis_index("s")
        tid = c * ns + s

        pltpu.sync_copy(sd_hbm.at[tid], sd)
        pltpu.sync_copy(w_hbm.at[tid], wv)

        zb = jnp.zeros((32,), jnp.bfloat16)

        @plsc.parallel_loop(0, ch)
        def zero_rows(r):
            for j in range(h // 32):
                rows[0][r, pl.ds(j * 32, 32)] = zb

        def zero_acc(r0, rpt):
            for i in range(rpt // ch):
                pltpu.sync_copy(rows[0], acc.at[pl.ds(r0 + i * ch, ch)])
            t = rpt % ch
            if t:
                pltpu.sync_copy(rows[0].at[pl.ds(0, t)],
                                acc.at[pl.ds(r0 + (rpt // ch) * ch, t)])

        @pl.when(s < 14)
        def _():
            zero_acc(s * 624, 624)

        @pl.when(s >= 14)
        def _():
            zero_acc(14 * 624 + (s - 14) * 632, 632)

        @plsc.parallel_loop(0, nk)
        def unpack_idx(k):
            for g in range(ch // 16):
                sl = pl.ds(g * 16, 16)
                v = sd[pl.ds(k * ch + g * 16, 16)]
                sidx[k, sl] = v & 0xFFFF
                didx[k, sl] = lax.shift_right_logical(v, 16)

        plsc.subcore_barrier()

        nbuf = len(rows)

        def start_gather(k, b):
            pltpu.async_copy(sup_hbm.at[sidx.at[k]], rows[b], gsem[b])

        def wait_gather(k, b):
            pltpu.make_async_copy(sup_hbm.at[sidx.at[k]], rows[b],
                                  gsem[b]).wait()

        def scale(k, b):
            @plsc.parallel_loop(0, ch // 16, unroll=2)
            def scale_body(g):
                wg = wv[pl.ds(k * ch + g * 16, 16)]
                for l in range(16):
                    wsplat = plsc.bitcast(jnp.broadcast_to(wg[l], (16,)),
                                          jnp.bfloat16)
                    e = g * 16 + l
                    for j in range(h // 32):
                        sl = pl.ds(j * 32, 32)
                        rows[b][e, sl] = rows[b][e, sl] * wsplat

        def start_scatter(k, b):
            pltpu.async_copy(rows[b], acc.at[didx.at[k]], ssem[b], add=True)

        def wait_scatter(k, b):
            pltpu.make_async_copy(rows[b], acc.at[didx.at[k]],
                                  ssem[b]).wait()

        for b in range(nbuf):
            start_gather(b, b)

        nq, rem = divmod(nk, nbuf)

        def ring_body(q, carry):
            kx = q * nbuf
            for b in range(nbuf):
                wait_gather(kx + b, b)
                scale(kx + b, b)
                start_scatter(kx + b, b)
            for b in range(nbuf):
                wait_scatter(kx + b, b)

                @pl.when(kx + b + nbuf < nk)
                def _(b=b):
                    start_gather(kx + b + nbuf, b)

            return carry

        lax.fori_loop(0, nq, ring_body, 0)
        for b in range(rem):
            kx = nq * nbuf + b
            wait_gather(kx, b)
            scale(kx, b)
            start_scatter(kx, b)
        for b in range(rem):
            wait_scatter(nq * nbuf + b, b)
        plsc.subcore_barrier()

        @pl.when(s < 14)
        def _():
            r0 = s * 624
            pltpu.sync_copy(acc.at[pl.ds(r0, 624)],
                            out_hbm.at[c, pl.ds(r0, 624)])

        @pl.when(s >= 14)
        def _():
            r0 = 14 * 624 + (s - 14) * 632
            pltpu.sync_copy(acc.at[pl.ds(r0, 632)],
                            out_hbm.at[c, pl.ds(r0, 632)])

    return agg(sup, sd_hbm_arr, w_hbm_arr)


def kernel(inputs, edge_index, edge_weight, weight, bias):
    n, d_in = inputs.shape
    e = edge_index.shape[1]
    d_out = weight.shape[1]

    nw = 32
    ch = 80
    assert e % (nw * ch) == 0
    nk = e // (nw * ch)

    xbf = inputs.astype(jnp.bfloat16)

    sd = (edge_index[0] | (edge_index[1] << 16)).reshape(nw, nk * ch)
    wb = lax.bitcast_convert_type(edge_weight.astype(jnp.bfloat16),
                                  jnp.uint16).astype(jnp.int32)
    wbits = (wb | (wb << 16)).reshape(nw, nk * ch)

    partials = _sc_aggregate(xbf, sd, wbits, nk, ch)

    return _matmul_bias(partials, weight, bias.reshape(1, d_out), n)
